# baseline (speedup 1.0000x reference)
import functools

import jax
import jax.numpy as jnp
from jax import lax
from jax.experimental import pallas as pl
from jax.experimental.pallas import tpu as pltpu
from jax.experimental.pallas import tpu_sc as plsc

_EMBED = 64
_NC = 2
_NS = 16
_NW = _NC * _NS
_BLK = 128
_L = 16


def _make_kernel(batch, hist):
    r_per_w = batch // _NW
    nrb = r_per_w // _BLK
    mesh = plsc.VectorSubcoreMesh(core_axis_name="c", subcore_axis_name="s")

    @functools.partial(
        pl.kernel,
        out_type=jax.ShapeDtypeStruct((hist, _EMBED, batch), jnp.float32),
        mesh=mesh,
        scratch_types=[
            pltpu.VMEM((2, r_per_w), jnp.int32),
            pltpu.VMEM((2, r_per_w), jnp.int32),
            pltpu.VMEM((2, r_per_w), jnp.int32),
            pltpu.VMEM((nrb, _BLK, 2 * _EMBED), jnp.float32),
            pltpu.VMEM((nrb, _EMBED, _BLK), jnp.float32),
            pltpu.SemaphoreType.DMA,
            pltpu.SemaphoreType.DMA,
            pltpu.SemaphoreType.DMA,
        ],
        compiler_params=pltpu.CompilerParams(needs_layout_passes=False),
    )
    def body(xt_hbm, tab_hbm, out_hbm, idx_v, pidx_v, hoff_v, rows_v, ot_v,
             isem, gsem, wsem):
        wid = lax.axis_index("s") * _NC + lax.axis_index("c")
        r0 = wid * r_per_w

        def fire_idx(h, p):
            pltpu.make_async_copy(
                xt_hbm.at[h, pl.ds(r0, r_per_w)], idx_v.at[p], isem).start()

        def wait_idx(h, p):
            pltpu.make_async_copy(
                xt_hbm.at[h, pl.ds(r0, r_per_w)], idx_v.at[p], isem).wait()

        def compute_pidx(p):
            for i in range(r_per_w // _L):
                v = idx_v[p, pl.ds(i * _L, _L)]
                pidx_v[p, pl.ds(i * _L, _L)] = lax.shift_right_logical(v, 1)
                hoff_v[p, pl.ds(i * _L, _L)] = lax.shift_left(
                    lax.bitwise_and(v, 1), 6)

        def fire_gather(p, rb):
            pltpu.make_async_copy(
                tab_hbm.at[pidx_v.at[p, pl.ds(rb * _BLK, _BLK)]],
                rows_v.at[rb], gsem).start()

        def wait_gather(p, rb):
            pltpu.make_async_copy(
                tab_hbm.at[pidx_v.at[p, pl.ds(rb * _BLK, _BLK)]],
                rows_v.at[rb], gsem).wait()

        def out_copy(h, rb):
            return pltpu.make_async_copy(
                ot_v.at[rb], out_hbm.at[h, :, pl.ds(r0 + rb * _BLK, _BLK)],
                wsem)

        rowids = [lax.iota(jnp.int32, _L) + g * _L for g in range(_BLK // _L)]

        def transpose_block(p, rb):
            hvec = [hoff_v[p, pl.ds(rb * _BLK + g * _L, _L)]
                    for g in range(_BLK // _L)]

            @pl.loop(0, _EMBED, step=8)
            def _e(e0):
                for de in range(8):
                    e = e0 + de
                    for g in range(_BLK // _L):
                        vals = plsc.load_gather(
                            rows_v.at[rb], [rowids[g], hvec[g] + e])
                        ot_v[rb, e, pl.ds(g * _L, _L)] = vals

        fire_idx(0, 0)
        wait_idx(0, 0)
        compute_pidx(0)
        fire_idx(1, 1)
        for rb in range(nrb):
            fire_gather(0, rb)

        @pl.loop(0, hist)
        def _h(h):
            p = lax.rem(h, 2)
            q = 1 - p

            @pl.when(h + 1 < hist)
            def _stage():
                wait_idx(h + 1, q)
                compute_pidx(q)

                @pl.when(h + 2 < hist)
                def _pf():
                    fire_idx(h + 2, p)

            for rb in range(nrb):
                wait_gather(p, rb)

                @pl.when(h > 0)
                def _drain():
                    out_copy(h, rb).wait()

                transpose_block(p, rb)
                out_copy(h, rb).start()

                @pl.when(h + 1 < hist)
                def _next():
                    fire_gather(q, rb)

        for rb in range(nrb):
            out_copy(hist - 1, rb).wait()

    return body


def kernel(x, vectors):
    b, h = x.shape
    xt = jnp.transpose(x)
    tab2 = jnp.reshape(vectors, (vectors.shape[0] // 2, 2 * _EMBED))
    out_t = _make_kernel(b, h)(xt, tab2)
    return jnp.transpose(out_t, (2, 0, 1))


# speedup vs baseline: 1.2841x; 1.2841x over previous
import functools

import jax
import jax.numpy as jnp
from jax import lax
from jax.experimental import pallas as pl
from jax.experimental.pallas import tpu as pltpu
from jax.experimental.pallas import tpu_sc as plsc

_EMBED = 64
_NC = 2
_NS = 16
_NW = _NC * _NS
_L = 16
_B = 256
_NBUF = 2


def _make_kernel(n):
    n_per_w = n // _NW
    nblk = n_per_w // _B
    mesh = plsc.VectorSubcoreMesh(core_axis_name="c", subcore_axis_name="s")

    @functools.partial(
        pl.kernel,
        out_type=jax.ShapeDtypeStruct((n * _EMBED // 128, 128), jnp.float32),
        mesh=mesh,
        scratch_types=[
            pltpu.VMEM((_NBUF, _B), jnp.int32),
            pltpu.VMEM((_NBUF, 2, _B // 2), jnp.int32),
            pltpu.VMEM((_NBUF, _B, 2 * _EMBED), jnp.float32),
            pltpu.VMEM((_NBUF, _B // 2, 2 * _EMBED), jnp.float32),
            pltpu.SemaphoreType.DMA,
            pltpu.SemaphoreType.DMA,
        ],
        compiler_params=pltpu.CompilerParams(needs_layout_passes=False),
    )
    def body(xf_hbm, tab_hbm, out_hbm, idx_v, pidx_v, rows_v, half_v,
             gsem, wsem):
        wid = lax.axis_index("s") * _NC + lax.axis_index("c")
        k0w = wid * n_per_w

        def prep_gather(k, b):
            pltpu.sync_copy(xf_hbm.at[pl.ds(pl.multiple_of(k0w + k * _B, 256), _B)], idx_v.at[b])
            for i in range(_B // _L):
                v = idx_v[b, pl.ds(i * _L, _L)]
                pidx_v[b, i // 8, pl.ds((i % 8) * _L, _L)] = (
                    lax.shift_right_logical(v, 1))
            for g in range(2):
                pltpu.make_async_copy(
                    tab_hbm.at[pidx_v.at[b, g]],
                    rows_v.at[b, pl.ds(g * (_B // 2), _B // 2)], gsem).start()

        def wait_gather(b):
            for g in range(2):
                pltpu.make_async_copy(
                    tab_hbm.at[pidx_v.at[b, g]],
                    rows_v.at[b, pl.ds(g * (_B // 2), _B // 2)], gsem).wait()

        def extract(b):
            @pl.loop(0, _B, step=_L)
            def _j(j0):
                hvec = lax.shift_left(
                    lax.bitwise_and(idx_v[b, pl.ds(j0, _L)], 1), 6)
                for kk in range(_L):
                    j = j0 + kk
                    h = hvec[kk]
                    jh = lax.shift_right_logical(j, 1)
                    jo = (kk % 2) * _EMBED
                    for q in range(_EMBED // _L):
                        half_v[b, jh, pl.ds(jo + q * _L, _L)] = (
                            rows_v[b, j, pl.ds(h + q * _L, _L)])

        def out_copy(k, b):
            off = pl.multiple_of((k0w + k * _B) // 2, 128)
            return pltpu.make_async_copy(
                half_v.at[b], out_hbm.at[pl.ds(off, _B // 2)], wsem)

        for b in range(_NBUF):
            prep_gather(b, b)

        @pl.loop(0, nblk, step=_NBUF)
        def _steady(k0):
            for b in range(_NBUF):
                k = k0 + b
                wait_gather(b)

                @pl.when(k >= _NBUF)
                def _drain():
                    out_copy(k, b).wait()

                extract(b)
                out_copy(k, b).start()

                @pl.when(k + _NBUF < nblk)
                def _next():
                    prep_gather(k + _NBUF, b)

        for b in range(_NBUF):
            out_copy(nblk - _NBUF + b, b).wait()

    return body


def kernel(x, vectors):
    b, h = x.shape
    n = b * h
    xf = jnp.reshape(x, (n,))
    tab2 = jnp.reshape(vectors, (vectors.shape[0] // 2, 2 * _EMBED))
    out2 = _make_kernel(n)(xf, tab2)
    return jnp.reshape(out2, (b, h, _EMBED))
